# baseline (speedup 1.0000x reference)
import functools

import jax
import jax.numpy as jnp
from jax import lax
from jax.experimental import pallas as pl
from jax.experimental.pallas import tpu as pltpu
from jax.experimental.pallas import tpu_sc as plsc

_THRESHOLD = 0.9952
_LANES = 16


def _tree_sum(vs):
    vs = list(vs)
    while len(vs) > 1:
        h = len(vs) // 2
        vs = [vs[i] + vs[i + h] for i in range(h)]
    return vs[0]


def kernel(input, filt):
    B, T = input.shape
    F = filt.shape[0]
    n_steps = T - F
    n_blocks = n_steps // F
    rows_per_worker = _LANES
    n_workers = B // rows_per_worker
    chunk = rows_per_worker * T

    info = plsc.get_sparse_core_info()
    nc = info.num_cores
    mesh = plsc.VectorSubcoreMesh(core_axis_name="c", subcore_axis_name="s")

    @functools.partial(
        pl.kernel,
        mesh=mesh,
        compiler_params=pltpu.CompilerParams(needs_layout_passes=False),
        out_type=jax.ShapeDtypeStruct((B * T,), jnp.float32),
        scratch_types=[
            pltpu.VMEM((chunk,), jnp.float32),
            pltpu.VMEM((chunk,), jnp.float32),
            pltpu.VMEM((F,), jnp.float32),
        ],
    )
    def bsa(x_hbm, filt_hbm, out_hbm, x_v, out_v, filt_v):
        wid = lax.axis_index("s") * nc + lax.axis_index("c")

        @pl.when(wid < n_workers)
        def _():
            base = wid * chunk
            pltpu.sync_copy(x_hbm.at[pl.ds(base, chunk)], x_v)
            pltpu.sync_copy(filt_hbm, filt_v)

            row_off = lax.iota(jnp.int32, _LANES) * T
            thr = jnp.float32(_THRESHOLD)
            one_v = jnp.ones((_LANES,), jnp.float32)
            zero_v = jnp.zeros((_LANES,), jnp.float32)
            fvec = filt_v[...]
            fb = [jnp.full((_LANES,), fvec[k]) for k in range(F)]

            ws = [plsc.load_gather(x_v, [row_off + k]) for k in range(F)]

            def block(jb, carry):
                ws = list(carry)
                ibase = row_off + jb * F
                xnews = [plsc.load_gather(x_v, [ibase + (k + F)]) for k in range(F)]
                for k in range(F):
                    xnew = xnews[k]
                    d1 = [ws[p] - fb[p] for p in range(F)]
                    e1 = jnp.abs(_tree_sum(d1))
                    e2 = jnp.abs(_tree_sum(ws)) * thr
                    m = e1 <= e2
                    spike = jnp.where(m, one_v, zero_v)
                    plsc.store_scatter(out_v, [ibase + k], spike)
                    shifted = ws[1:] + [xnew]
                    ws = [
                        jnp.where(m, shifted[p] - fb[p], shifted[p])
                        for p in range(F)
                    ]
                return tuple(ws)

            lax.fori_loop(0, n_blocks, block, tuple(ws))

            for j in range(n_steps, T):
                plsc.store_scatter(out_v, [row_off + j], zero_v)

            pltpu.sync_copy(out_v, out_hbm.at[pl.ds(base, chunk)])

    out_flat = bsa(input.reshape(B * T), filt)
    return out_flat.reshape(B, T)


# speedup vs baseline: 1.1929x; 1.1929x over previous
import functools

import jax
import jax.numpy as jnp
from jax import lax
from jax.experimental import pallas as pl
from jax.experimental.pallas import tpu as pltpu
from jax.experimental.pallas import tpu_sc as plsc

_THRESHOLD = 0.9952
_LANES = 16


def kernel(input, filt):
    B, T = input.shape
    F = filt.shape[0]
    n_steps = T - F
    n_blocks = n_steps // F

    info = plsc.get_sparse_core_info()
    nc, ns = info.num_cores, info.num_subcores
    n_workers = nc * ns
    rows_per_worker = B // n_workers
    chunk = rows_per_worker * T
    nv = F // rows_per_worker

    mesh = plsc.VectorSubcoreMesh(core_axis_name="c", subcore_axis_name="s")

    @functools.partial(
        pl.kernel,
        mesh=mesh,
        compiler_params=pltpu.CompilerParams(needs_layout_passes=False),
        out_type=jax.ShapeDtypeStruct((B * T,), jnp.float32),
        scratch_types=[
            pltpu.VMEM((chunk,), jnp.float32),
            pltpu.VMEM((chunk,), jnp.float32),
            pltpu.VMEM((F,), jnp.float32),
            pltpu.VMEM((F * nv * _LANES,), jnp.float32),
        ],
    )
    def bsa(x_hbm, filt_hbm, out_hbm, x_v, out_v, filt_v, ftab_v):
        wid = lax.axis_index("s") * nc + lax.axis_index("c")
        base = wid * chunk
        pltpu.sync_copy(x_hbm.at[pl.ds(base, chunk)], x_v)
        pltpu.sync_copy(filt_hbm, filt_v)

        lane = lax.iota(jnp.int32, _LANES)
        row4 = (lane >> 2) * T
        pos4 = lane & 3
        thr = jnp.float32(_THRESHOLD)
        one_v = jnp.ones((_LANES,), jnp.float32)
        zero_v = jnp.zeros((_LANES,), jnp.float32)
        lane0 = pos4 == 0
        posmask = [pos4 == i for i in range(rows_per_worker)]
        xor2 = jax.lax.bitwise_xor(lane, 2)
        xor1 = jax.lax.bitwise_xor(lane, 1)

        def shuf(vv, idxv):
            return jnp.take_along_axis(vv, idxv, axis=0,
                                       mode="promise_in_bounds")

        def tree(a):
            p1a = a[0] + a[2]
            p1b = a[1] + a[3]
            p2 = p1a + p1b
            p3 = p2 + shuf(p2, xor2)
            return p3 + shuf(p3, xor1)

        for r in range(F):
            for i in range(nv):
                idx = (pos4 + ((4 * i - r) % F)) & (F - 1)
                ftab_v[pl.ds((r * nv + i) * _LANES, _LANES)] = (
                    plsc.load_gather(filt_v, [idx]))

        v = [plsc.load_gather(x_v, [row4 + pos4 + 4 * i]) for i in range(nv)]
        taps = [ftab_v[pl.ds(i * _LANES, _LANES)] for i in range(nv)]

        def block(jb, carry):
            v = list(carry[:nv])
            taps = list(carry[nv:])
            bvec0 = row4 + jb * F
            for k in range(F):
                idx_out = bvec0 + k
                xnew = plsc.load_gather(x_v, [idx_out + F])
                d1 = [v[i] - taps[i] for i in range(nv)]
                e1 = jnp.abs(tree(d1))
                e2 = jnp.abs(tree(v)) * thr
                m = e1 <= e2
                spike = jnp.where(m, one_v, zero_v)
                plsc.store_scatter(out_v, [idx_out], spike, mask=lane0)
                ke, le = k // 4, k % 4
                v[ke] = jnp.where(posmask[le], xnew, v[ke])
                rn = (k + 1) % F
                taps = [ftab_v[pl.ds((rn * nv + i) * _LANES, _LANES)]
                        for i in range(nv)]
                v = [jnp.where(m, v[i] - taps[i], v[i]) for i in range(nv)]
            return (*v, *taps)

        lax.fori_loop(0, n_blocks, block, (*v, *taps))

        for j in range(n_steps, T):
            plsc.store_scatter(out_v, [row4 + j], zero_v, mask=lane0)

        pltpu.sync_copy(out_v, out_hbm.at[pl.ds(base, chunk)])

    out_flat = bsa(input.reshape(B * T), filt)
    return out_flat.reshape(B, T)
